# Initial kernel scaffold; baseline (speedup 1.0000x reference)
import jax
import jax.numpy as jnp
from jax.experimental import pallas as pl


def kernel(item, cate, hist_item, hist_cate, mask, table, W1, b1, a1, W2, b2, a2, W3, b3):
    raise NotImplementedError("write your pallas kernel here")


# speedup vs baseline: 14.6276x; 14.6276x over previous
import functools

import jax
import jax.numpy as jnp
from jax import lax
from jax.experimental import pallas as pl
from jax.experimental.pallas import tpu as pltpu
from jax.experimental.pallas import tpu_sc as plsc

_B = 16384
_L = 200
_D = 32
_NC = 2
_NS = 16
_NW = _NC * _NS
_BPW = _B // _NW
_CB = 4
_NCHUNK = _BPW // _CB
_IDXROWS = 4 * _CB


def _sc_pool_body(table, item2d, cate2d, histi2, histc2, mask_f,
                  e_item, e_cate, h_item, h_cate,
                  ivec, rowbuf, idx_h, mask_v, rows_h, hbuf_i, hbuf_c, sem):
    wid = lax.axis_index("s") * _NC + lax.axis_index("c")
    base = wid * _BPW

    for src2d, out in ((item2d, e_item), (cate2d, e_cate)):
        pltpu.sync_copy(src2d.at[pl.ds(wid * 4, 4)], ivec)
        cps = [pltpu.async_copy(table.at[ivec.at[j]],
                                rowbuf.at[pl.ds(j * 128, 128)], sem)
               for j in range(4)]
        for cp in cps:
            cp.wait()
        pltpu.sync_copy(rowbuf, out.at[pl.ds(base, _BPW)])

    def chunk_body(c, carry):
        b0 = base + c * _CB
        pltpu.sync_copy(histi2.at[pl.ds(2 * b0, 2 * _CB)],
                        idx_h.at[pl.ds(0, 2 * _CB)])
        pltpu.sync_copy(histc2.at[pl.ds(2 * b0, 2 * _CB)],
                        idx_h.at[pl.ds(2 * _CB, 2 * _CB)])
        pltpu.sync_copy(mask_f.at[pl.ds(b0 * _L, _CB * _L)], mask_v)
        cps = [pltpu.async_copy(table.at[idx_h.at[j]],
                                rows_h.at[pl.ds(j * 100, 100)], sem)
               for j in range(_IDXROWS)]
        for cp in cps:
            cp.wait()
        for i in range(_CB):
            def accum(base_f, accs, js):
                a0, a1, a2, a3 = accs
                mvec = mask_v[pl.ds(base_f, 16)]
                for j in js:
                    m = mvec[j]
                    ri = base_f + j
                    rc = _CB * _L + base_f + j
                    a0 = a0 + rows_h[ri, pl.ds(0, 16)] * m
                    a1 = a1 + rows_h[ri, pl.ds(16, 16)] * m
                    a2 = a2 + rows_h[rc, pl.ds(0, 16)] * m
                    a3 = a3 + rows_h[rc, pl.ds(16, 16)] * m
                return (a0, a1, a2, a3)

            def g_body(g, accs, i=i):
                return accum(i * _L + g * 16, accs, range(16))
            z = jnp.zeros((16,), jnp.float32)
            accs = lax.fori_loop(0, _L // 16, g_body, (z, z, z, z))
            a0, a1, a2, a3 = accum(i * _L + (_L - 16), accs, range(8, 16))
            hbuf_i[i, pl.ds(0, 16)] = a0
            hbuf_i[i, pl.ds(16, 16)] = a1
            hbuf_c[i, pl.ds(0, 16)] = a2
            hbuf_c[i, pl.ds(16, 16)] = a3
        pltpu.sync_copy(hbuf_i, h_item.at[pl.ds(b0, _CB)])
        pltpu.sync_copy(hbuf_c, h_cate.at[pl.ds(b0, _CB)])
        return carry

    lax.fori_loop(0, _NCHUNK, chunk_body, 0)


@jax.jit
def _sc_pool(table, item2d, cate2d, histi2, histc2, mask_f):
    f32 = jnp.float32
    out = jax.ShapeDtypeStruct((_B, _D), f32)
    k = functools.partial(
        pl.kernel,
        mesh=plsc.VectorSubcoreMesh(core_axis_name="c", subcore_axis_name="s"),
        out_type=[out, out, out, out],
        scratch_types=[
            pltpu.VMEM((4, 128), jnp.int32),
            pltpu.VMEM((_BPW, _D), f32),
            pltpu.VMEM((_IDXROWS, 100), jnp.int32),
            pltpu.VMEM((_CB * _L,), f32),
            pltpu.VMEM((2 * _CB * _L, _D), f32),
            pltpu.VMEM((_CB, 2 * 16), f32),
            pltpu.VMEM((_CB, 2 * 16), f32),
            pltpu.SemaphoreType.DMA,
        ],
        compiler_params=pltpu.CompilerParams(use_tc_tiling_on_sc=False),
    )(_sc_pool_body)
    return k(table, item2d, cate2d, histi2, histc2, mask_f)


def _mlp_body(ei, ec, hi, hc, w1, b1, a1, w2, b2, a2, w3, b3, out):
    f32 = jnp.float32
    x1 = (jnp.dot(ei[...], w1[pl.ds(0, 32), :], preferred_element_type=f32)
          + jnp.dot(ec[...], w1[pl.ds(32, 32), :], preferred_element_type=f32)
          + jnp.dot(hi[...], w1[pl.ds(64, 32), :], preferred_element_type=f32)
          + jnp.dot(hc[...], w1[pl.ds(96, 32), :], preferred_element_type=f32)
          + b1[...])
    x1 = jnp.where(x1 >= 0, x1, a1[0] * x1)
    x2 = jnp.dot(x1, w2[...], preferred_element_type=f32) + b2[...]
    x2 = jnp.where(x2 >= 0, x2, a2[0] * x2)
    out[...] = jnp.dot(x2, w3[...], preferred_element_type=f32) + b3[...]


_MBLK = 2048


@jax.jit
def _mlp(ei, ec, hi, hc, w1, b1, a1, w2, b2, a2, w3, b3):
    f32 = jnp.float32
    grid = (_B // _MBLK,)
    feat = lambda i: (i, 0)
    rep = lambda i: (0, 0)
    return pl.pallas_call(
        _mlp_body,
        grid=grid,
        in_specs=[
            pl.BlockSpec((_MBLK, _D), feat),
            pl.BlockSpec((_MBLK, _D), feat),
            pl.BlockSpec((_MBLK, _D), feat),
            pl.BlockSpec((_MBLK, _D), feat),
            pl.BlockSpec((128, 200), rep),
            pl.BlockSpec((1, 200), rep),
            pl.BlockSpec(memory_space=pltpu.SMEM),
            pl.BlockSpec((200, 80), rep),
            pl.BlockSpec((1, 80), rep),
            pl.BlockSpec(memory_space=pltpu.SMEM),
            pl.BlockSpec((80, 2), rep),
            pl.BlockSpec((1, 2), rep),
        ],
        out_specs=pl.BlockSpec((_MBLK, 2), feat),
        out_shape=jax.ShapeDtypeStruct((_B, 2), f32),
    )(ei, ec, hi, hc, w1, b1, a1, w2, b2, a2, w3, b3)


def kernel(item, cate, hist_item, hist_cate, mask, table,
           W1, b1, a1, W2, b2, a2, W3, b3):
    i32 = jnp.int32
    item2d = item.astype(i32).reshape(_B // 128, 128)
    cate2d = cate.astype(i32).reshape(_B // 128, 128)
    histi2 = hist_item.astype(i32).reshape(2 * _B, _L // 2)
    histc2 = hist_cate.astype(i32).reshape(2 * _B, _L // 2)
    mask_f = mask.reshape(_B * _L)
    e_item, e_cate, h_item, h_cate = _sc_pool(
        table, item2d, cate2d, histi2, histc2, mask_f)
    return _mlp(e_item, e_cate, h_item, h_cate,
                W1, b1.reshape(1, 200), a1, W2, b2.reshape(1, 80), a2,
                W3, b3.reshape(1, 2))


# speedup vs baseline: 18.2079x; 1.2448x over previous
import functools

import jax
import jax.numpy as jnp
from jax import lax
from jax.experimental import pallas as pl
from jax.experimental.pallas import tpu as pltpu
from jax.experimental.pallas import tpu_sc as plsc

_B = 16384
_L = 200
_D = 32
_NC = 2
_NS = 16
_NW = _NC * _NS
_BPW = _B // _NW
_CB = 4
_NCHUNK = _BPW // _CB
_IDXROWS = 4 * _CB


def _sc_pool_body(table, item2d, cate2d, histi2, histc2, mask_f,
                  e_item, e_cate, h_item, h_cate,
                  ivec, idx0, idx1, mask0, mask1, rows0, rows1,
                  hbuf_i, hbuf_c, sem0, sem1):
    wid = lax.axis_index("s") * _NC + lax.axis_index("c")
    base = wid * _BPW

    for src2d, out in ((item2d, e_item), (cate2d, e_cate)):
        pltpu.sync_copy(src2d.at[pl.ds(wid * 4, 4)], ivec)
        cps = [pltpu.async_copy(table.at[ivec.at[j]],
                                rows0.at[pl.ds(j * 128, 128)], sem0)
               for j in range(4)]
        for cp in cps:
            cp.wait()
        pltpu.sync_copy(rows0.at[pl.ds(0, _BPW)], out.at[pl.ds(base, _BPW)])

    def stage(cidx, idx_h, mask_v, rows_h, sem):
        b0 = base + cidx * _CB
        pltpu.sync_copy(histi2.at[pl.ds(2 * b0, 2 * _CB)],
                        idx_h.at[pl.ds(0, 2 * _CB)])
        pltpu.sync_copy(histc2.at[pl.ds(2 * b0, 2 * _CB)],
                        idx_h.at[pl.ds(2 * _CB, 2 * _CB)])
        pltpu.sync_copy(mask_f.at[pl.ds(b0 * _L, _CB * _L)], mask_v)
        for j in range(_IDXROWS):
            pltpu.async_copy(table.at[idx_h.at[j]],
                             rows_h.at[pl.ds(j * 100, 100)], sem)

    def drain(rows_h, sem):
        for j in range(_IDXROWS):
            pltpu.make_async_copy(table.at[pl.ds(0, 100)],
                                  rows_h.at[pl.ds(j * 100, 100)], sem).wait()

    def compute(cidx, idx_h, mask_v, rows_h):
        b0 = base + cidx * _CB
        for i in range(_CB):
            def accum(base_f, accs, js):
                a0, a1, a2, a3 = accs
                mvec = mask_v[pl.ds(base_f, 16)]
                for j in js:
                    m = mvec[j]
                    ri = base_f + j
                    rc = _CB * _L + base_f + j
                    a0 = a0 + rows_h[ri, pl.ds(0, 16)] * m
                    a1 = a1 + rows_h[ri, pl.ds(16, 16)] * m
                    a2 = a2 + rows_h[rc, pl.ds(0, 16)] * m
                    a3 = a3 + rows_h[rc, pl.ds(16, 16)] * m
                return (a0, a1, a2, a3)

            def g_body(g, accs, i=i):
                return accum(i * _L + g * 16, accs, range(16))
            z = jnp.zeros((16,), jnp.float32)
            accs = lax.fori_loop(0, _L // 16, g_body, (z, z, z, z))
            a0, a1, a2, a3 = accum(i * _L + (_L - 16), accs, range(8, 16))
            hbuf_i[i, pl.ds(0, 16)] = a0
            hbuf_i[i, pl.ds(16, 16)] = a1
            hbuf_c[i, pl.ds(0, 16)] = a2
            hbuf_c[i, pl.ds(16, 16)] = a3
        pltpu.sync_copy(hbuf_i, h_item.at[pl.ds(b0, _CB)])
        pltpu.sync_copy(hbuf_c, h_cate.at[pl.ds(b0, _CB)])

    stage(0, idx0, mask0, rows0, sem0)

    def outer(c2, carry):
        c = 2 * c2
        stage(c + 1, idx1, mask1, rows1, sem1)
        drain(rows0, sem0)
        compute(c, idx0, mask0, rows0)

        @pl.when(c2 < _NCHUNK // 2 - 1)
        def _():
            stage(c + 2, idx0, mask0, rows0, sem0)
        drain(rows1, sem1)
        compute(c + 1, idx1, mask1, rows1)
        return carry

    lax.fori_loop(0, _NCHUNK // 2, outer, 0)


@jax.jit
def _sc_pool(table, item2d, cate2d, histi2, histc2, mask_f):
    f32 = jnp.float32
    out = jax.ShapeDtypeStruct((_B, _D), f32)
    k = functools.partial(
        pl.kernel,
        mesh=plsc.VectorSubcoreMesh(core_axis_name="c", subcore_axis_name="s"),
        out_type=[out, out, out, out],
        scratch_types=[
            pltpu.VMEM((4, 128), jnp.int32),
            pltpu.VMEM((_IDXROWS, 100), jnp.int32),
            pltpu.VMEM((_IDXROWS, 100), jnp.int32),
            pltpu.VMEM((_CB * _L,), f32),
            pltpu.VMEM((_CB * _L,), f32),
            pltpu.VMEM((2 * _CB * _L, _D), f32),
            pltpu.VMEM((2 * _CB * _L, _D), f32),
            pltpu.VMEM((_CB, 2 * 16), f32),
            pltpu.VMEM((_CB, 2 * 16), f32),
            pltpu.SemaphoreType.DMA,
            pltpu.SemaphoreType.DMA,
        ],
        compiler_params=pltpu.CompilerParams(use_tc_tiling_on_sc=False),
    )(_sc_pool_body)
    return k(table, item2d, cate2d, histi2, histc2, mask_f)


def _mlp_body(ei, ec, hi, hc, w1, b1, a1, w2, b2, a2, w3, b3, out):
    f32 = jnp.float32
    x1 = (jnp.dot(ei[...], w1[pl.ds(0, 32), :], preferred_element_type=f32)
          + jnp.dot(ec[...], w1[pl.ds(32, 32), :], preferred_element_type=f32)
          + jnp.dot(hi[...], w1[pl.ds(64, 32), :], preferred_element_type=f32)
          + jnp.dot(hc[...], w1[pl.ds(96, 32), :], preferred_element_type=f32)
          + b1[...])
    x1 = jnp.where(x1 >= 0, x1, a1[0] * x1)
    x2 = jnp.dot(x1, w2[...], preferred_element_type=f32) + b2[...]
    x2 = jnp.where(x2 >= 0, x2, a2[0] * x2)
    out[...] = jnp.dot(x2, w3[...], preferred_element_type=f32) + b3[...]


_MBLK = 2048


@jax.jit
def _mlp(ei, ec, hi, hc, w1, b1, a1, w2, b2, a2, w3, b3):
    f32 = jnp.float32
    grid = (_B // _MBLK,)
    feat = lambda i: (i, 0)
    rep = lambda i: (0, 0)
    return pl.pallas_call(
        _mlp_body,
        grid=grid,
        in_specs=[
            pl.BlockSpec((_MBLK, _D), feat),
            pl.BlockSpec((_MBLK, _D), feat),
            pl.BlockSpec((_MBLK, _D), feat),
            pl.BlockSpec((_MBLK, _D), feat),
            pl.BlockSpec((128, 200), rep),
            pl.BlockSpec((1, 200), rep),
            pl.BlockSpec(memory_space=pltpu.SMEM),
            pl.BlockSpec((200, 80), rep),
            pl.BlockSpec((1, 80), rep),
            pl.BlockSpec(memory_space=pltpu.SMEM),
            pl.BlockSpec((80, 2), rep),
            pl.BlockSpec((1, 2), rep),
        ],
        out_specs=pl.BlockSpec((_MBLK, 2), feat),
        out_shape=jax.ShapeDtypeStruct((_B, 2), f32),
    )(ei, ec, hi, hc, w1, b1, a1, w2, b2, a2, w3, b3)


def kernel(item, cate, hist_item, hist_cate, mask, table,
           W1, b1, a1, W2, b2, a2, W3, b3):
    i32 = jnp.int32
    item2d = item.astype(i32).reshape(_B // 128, 128)
    cate2d = cate.astype(i32).reshape(_B // 128, 128)
    histi2 = hist_item.astype(i32).reshape(2 * _B, _L // 2)
    histc2 = hist_cate.astype(i32).reshape(2 * _B, _L // 2)
    mask_f = mask.reshape(_B * _L)
    e_item, e_cate, h_item, h_cate = _sc_pool(
        table, item2d, cate2d, histi2, histc2, mask_f)
    return _mlp(e_item, e_cate, h_item, h_cate,
                W1, b1.reshape(1, 200), a1, W2, b2.reshape(1, 80), a2,
                W3, b3.reshape(1, 2))
